# baseline (speedup 1.0000x reference)
import math

import jax
import jax.numpy as jnp
from jax.experimental import pallas as pl
from jax.experimental.pallas import tpu as pltpu

B, T = 2, 8
M = B * T
DEPTH = 1
HEADS = 4
NUM_CLASSES = 11
C_PAD = 128
SD = 32
HT_DIM, PR_DIM = 16, 16
DIM = 128
NUM_STREAMS = 3
IMU_IN, KP_IN, BBOX_IN = 24, 32, 8
CONV_K = 5
CONV_PAD = 2
LN_EPS = 1e-5

_CONF_KEYS = ("ff1_g", "ff1_b", "ff1_w1", "ff1_b1", "ff1_w2", "ff1_b2",
              "at_g", "at_b", "w_qkv", "b_qkv", "w_o", "b_o",
              "cv_g", "cv_b", "pw1_w", "pw1_b", "dw_w", "dw_s", "dw_sh",
              "pw2_w", "pw2_b",
              "ff2_g", "ff2_b", "ff2_w1", "ff2_b1", "ff2_w2", "ff2_b2",
              "lno_g", "lno_b")
_LSTM_KEYS = ("wih0", "b0", "whh0f", "whh0b", "wih1", "b1", "whh1f", "whh1b")
_INNER_KEYS = _LSTM_KEYS + ("p1_w", "p1_b", "p2_w", "p2_b")

_CONF_MATS = ("ff1_w1", "ff1_w2", "w_qkv", "w_o", "pw1_w", "pw2_w",
              "ff2_w1", "ff2_w2")
_LSTM_MATS = ("wih0", "whh0f", "whh0b", "wih1", "whh1f", "whh1b")


def _conf_small_widths(d):
    return (("ff1_g", d), ("ff1_b", d), ("ff1_b1", 4 * d), ("ff1_b2", d),
            ("at_g", d), ("at_b", d), ("b_qkv", 3 * d), ("b_o", d),
            ("cv_g", d), ("cv_b", d), ("pw1_b", 2 * d), ("dw_w", 5 * d),
            ("dw_s", d), ("dw_sh", d), ("pw2_b", d),
            ("ff2_g", d), ("ff2_b", d), ("ff2_b1", 4 * d), ("ff2_b2", d),
            ("lno_g", d), ("lno_b", d))


_SPACK = (("emb_b", SD),) + _conf_small_widths(SD) + (
    ("b0", 8 * SD), ("b1", 8 * SD), ("p1_b", C_PAD), ("p2_b", SD))

_FPACK = _conf_small_widths(DIM) + (
    ("fus_b1", 4 * DIM), ("fus_b2", DIM), ("ln_g", DIM), ("ln_b", DIM),
    ("fb0", 8 * DIM), ("fb1", 8 * DIM), ("head_b", C_PAD))


def _pack_offsets(spec):
    offs, off = {}, 0
    for k, wdt in spec:
        offs[k] = (off, wdt)
        off += wdt
    return offs


_SOFF = _pack_offsets(_SPACK)
_FOFF = _pack_offsets(_FPACK)


def _tree_template():
    conf = lambda: {k: 0 for k in _CONF_KEYS}
    inner = {k: 0 for k in _INNER_KEYS}
    streams = {"emb_w": 0, "emb_b": 0, "pos": 0,
               "blocks": [conf() for _ in range(DEPTH)], "inner": inner}
    return {"streams": streams, "emb_ht": 0, "emb_printer": 0,
            "fusion": {"w1": 0, "b1": 0, "w2": 0, "b2": 0},
            "layers": [conf() for _ in range(DEPTH)],
            "final": {"ln_g": 0, "ln_b": 0,
                      "lstm": {k: 0 for k in _LSTM_KEYS},
                      "head_w": 0, "head_b": 0}}


def _silu(x):
    return x * jax.nn.sigmoid(x)


def _ln(x, g, b):
    mu = jnp.mean(x, axis=-1, keepdims=True)
    var = jnp.mean(jnp.square(x - mu), axis=-1, keepdims=True)
    return (x - mu) * jax.lax.rsqrt(var + LN_EPS) * g + b


def _bmm(x, w):
    return jnp.einsum("smd,sdk->smk", x, w,
                      preferred_element_type=jnp.float32)


def _bmmh(x, w):
    return jnp.einsum("smd,sdk->smk", x.astype(jnp.bfloat16),
                      w.astype(jnp.bfloat16),
                      preferred_element_type=jnp.float32)


def _cell(g, c, hd):
    sg = jax.nn.sigmoid(g)
    tg = jnp.tanh(g[..., 2 * hd:3 * hd])
    c = sg[..., 1 * hd:2 * hd] * c + sg[..., 0 * hd:1 * hd] * tg
    return sg[..., 3 * hd:4 * hd] * jnp.tanh(c), c


def _bidir(xg, whf, whb, out_ref, hd):
    s = xg.shape[0]
    whf = whf.astype(jnp.bfloat16)
    whb = whb.astype(jnp.bfloat16)
    hf = jnp.zeros((s, B, hd), jnp.float32)
    cf = hf
    hb = hf
    cb = hf
    for u in range(T):
        v = T - 1 - u
        gf = xg[:, :, u, :4 * hd] + jnp.einsum(
            "sbh,shk->sbk", hf.astype(jnp.bfloat16), whf,
            preferred_element_type=jnp.float32)
        gb = xg[:, :, v, 4 * hd:] + jnp.einsum(
            "sbh,shk->sbk", hb.astype(jnp.bfloat16), whb,
            preferred_element_type=jnp.float32)
        hf, cf = _cell(gf, cf, hd)
        hb, cb = _cell(gb, cb, hd)
        out_ref[:, :, u:u + 1, 0:hd] = hf[:, :, None, :]
        out_ref[:, :, v:v + 1, hd:2 * hd] = hb[:, :, None, :]


def _bilstm2(x, w, hd, s0_ref, s1_ref):
    s = x.shape[0]
    xg0 = (_bmmh(x, w["wih0"]) + w["b0"]).reshape(s, B, T, 8 * hd)
    _bidir(xg0, w["whh0f"], w["whh0b"], s0_ref, hd)
    h0 = s0_ref[...].reshape(s, M, 2 * hd)
    xg1 = (_bmmh(h0, w["wih1"]) + w["b1"]).reshape(s, B, T, 8 * hd)
    _bidir(xg1, w["whh1f"], w["whh1b"], s1_ref, hd)
    return s1_ref[...].reshape(s, M, 2 * hd)


def _conformer(x, w, d):
    s = x.shape[0]
    dh = d // HEADS
    scale = 1.0 / math.sqrt(dh)

    y = _ln(x, w["ff1_g"], w["ff1_b"])
    y = _silu(_bmm(y, w["ff1_w1"]) + w["ff1_b1"])
    x = x + 0.5 * (_bmm(y, w["ff1_w2"]) + w["ff1_b2"])

    y = _ln(x, w["at_g"], w["at_b"])
    qkv = (_bmm(y, w["w_qkv"]) + w["b_qkv"]).reshape(s * B, T, 3 * d)
    q = jnp.concatenate([qkv[..., h * dh:(h + 1) * dh]
                         for h in range(HEADS)], axis=0)
    k = jnp.concatenate([qkv[..., d + h * dh:d + (h + 1) * dh]
                         for h in range(HEADS)], axis=0)
    v = jnp.concatenate([qkv[..., 2 * d + h * dh:2 * d + (h + 1) * dh]
                         for h in range(HEADS)], axis=0)
    sc = jnp.einsum("btd,bud->btu", q, k,
                    preferred_element_type=jnp.float32) * scale
    sc = sc - jnp.max(sc, axis=-1, keepdims=True)
    p = jnp.exp(sc)
    p = p / jnp.sum(p, axis=-1, keepdims=True)
    o = jnp.einsum("btu,bud->btd", p, v,
                   preferred_element_type=jnp.float32)
    att = jnp.concatenate([o[h * s * B:(h + 1) * s * B]
                           for h in range(HEADS)], axis=-1).reshape(s, M, d)
    x = x + _bmm(att, w["w_o"]) + w["b_o"]

    y = _ln(x, w["cv_g"], w["cv_b"])
    y = _bmm(y, w["pw1_w"]) + w["pw1_b"]
    y = y[..., :d] * jax.nn.sigmoid(y[..., d:])
    y4 = y.reshape(s, B, T, d)
    zpad = jnp.zeros((s, B, CONV_PAD, d), jnp.float32)
    yp = jnp.concatenate([zpad, y4, zpad], axis=2)
    acc = jnp.zeros((s, B, T, d), jnp.float32)
    for kk in range(CONV_K):
        tap = w["dw_w"][:, :, kk * d:(kk + 1) * d]
        acc = acc + yp[:, :, kk:kk + T, :] * tap[:, :, None, :]
    y4 = _silu(acc * w["dw_s"][:, None] + w["dw_sh"][:, None])
    x = x + _bmm(y4.reshape(s, M, d), w["pw2_w"]) + w["pw2_b"]

    y = _ln(x, w["ff2_g"], w["ff2_b"])
    y = _silu(_bmm(y, w["ff2_w1"]) + w["ff2_b1"])
    x = x + 0.5 * (_bmm(y, w["ff2_w2"]) + w["ff2_b2"])

    return _ln(x, w["lno_g"], w["lno_b"])


def _mega_kernel(*refs):
    (imu_r, kp_r, bb_r, htf_r, prf_r,
     emb_w, pos, eht, epr, spack_r, fpack_r) = refs[:11]
    i = 11
    smats = {k: refs[i + j] for j, k in enumerate(_CONF_MATS)}
    i += len(_CONF_MATS)
    imats = {k: refs[i + j]
             for j, k in enumerate(_LSTM_MATS + ("p1_w", "p2_w"))}
    i += 8
    fus_w1, fus_w2 = refs[i], refs[i + 1]
    i += 2
    fmats = {k: refs[i + j] for j, k in enumerate(_CONF_MATS)}
    i += len(_CONF_MATS)
    flmats = {k: refs[i + j] for j, k in enumerate(_LSTM_MATS)}
    i += len(_LSTM_MATS)
    head_w = refs[i]
    out_ref, inner_ref = refs[i + 1], refs[i + 2]
    s0_ref, s1_ref, f0_ref, f1_ref = refs[i + 3:i + 7]

    spack = spack_r[...]
    fpack = fpack_r[...]
    sp = {k: spack[:, :, o:o + wd] for k, (o, wd) in _SOFF.items()}
    fp = {k: fpack[:, o:o + wd] for k, (o, wd) in _FOFF.items()}

    pos_t = pos[:, :T, :]
    pos_m = jnp.concatenate([pos_t, pos_t], axis=1)

    def _embed(chunks, w_rows, widths):
        acc = jnp.zeros((M, SD), jnp.float32)
        off = 0
        for ch, wd in zip(chunks, widths):
            acc = acc + jnp.dot(ch.reshape(M, wd), w_rows[off:off + wd, :],
                                preferred_element_type=jnp.float32)
            off += wd
        return acc

    x0 = _embed([imu_r[:, :, f, :] for f in range(4)],
                emb_w[0], [6] * 4)
    x1 = _embed([kp_r[:, :, i, j, :] for i in range(2) for j in range(2)],
                emb_w[1], [8] * 4)
    x2 = _embed([bb_r[:, :, i, :] for i in range(2)],
                emb_w[2], [4] * 2)
    x = jnp.stack([x0, x1, x2], axis=0) + sp["emb_b"] + pos_m

    sw = {k: smats[k][...] for k in _CONF_MATS}
    sw.update({k: sp[k] for k, _ in _conf_small_widths(SD)})
    x = _conformer(x, sw, SD)
    iw = {k: imats[k][...] for k in _LSTM_MATS + ("p1_w", "p2_w")}
    iw.update({k: sp[k] for k in ("b0", "b1", "p1_b", "p2_b")})
    h = _bilstm2(x, iw, SD, s0_ref, s1_ref)
    logits = _bmm(h, iw["p1_w"]) + iw["p1_b"]
    lane = jax.lax.broadcasted_iota(jnp.int32, logits.shape, 2)
    valid = lane < NUM_CLASSES
    mx = jnp.max(jnp.where(valid, logits, -jnp.inf), axis=-1, keepdims=True)
    e = jnp.where(valid, jnp.exp(logits - mx), 0.0)
    p = e / jnp.sum(e, axis=-1, keepdims=True)
    x = x + _bmm(p, iw["p2_w"]) + iw["p2_b"]
    inner_ref[...] = jnp.mean(logits, axis=0)[:, :NUM_CLASSES].reshape(
        B, T, NUM_CLASSES)

    bsel = (jax.lax.broadcasted_iota(jnp.int32, (M, B), 0) // T
            == jax.lax.broadcasted_iota(jnp.int32, (M, B), 1)
            ).astype(jnp.float32)
    tmask = (jax.lax.broadcasted_iota(jnp.int32, (M, T), 0) % T
             == jax.lax.broadcasted_iota(jnp.int32, (M, T), 1)
             ).astype(jnp.float32)
    htf = jnp.sum(jnp.dot(bsel, htf_r[...].astype(jnp.float32),
                          preferred_element_type=jnp.float32) * tmask,
                  axis=1, keepdims=True)
    prf = jnp.sum(jnp.dot(bsel, prf_r[...].astype(jnp.float32),
                          preferred_element_type=jnp.float32) * tmask,
                  axis=1, keepdims=True)
    e0, e1 = eht[0:1, :], eht[1:2, :]
    x_ht = e0 + htf * (e1 - e0)
    q0, q1 = epr[0:1, :], epr[1:2, :]
    x_pr = q0 + prf * (q1 - q0)

    xf = jnp.concatenate([x[0], x[1], x_ht, x_pr, x[2]], axis=-1)
    y = _silu(jnp.dot(xf, fus_w1[...],
                      preferred_element_type=jnp.float32) + fp["fus_b1"])
    xf = jnp.dot(y, fus_w2[...],
                 preferred_element_type=jnp.float32) + fp["fus_b2"]

    fw = {k: fmats[k][...] for k in _CONF_MATS}
    fw.update({k: fp[k][:, None, :] for k, _ in _conf_small_widths(DIM)})
    xf = _conformer(xf[None], fw, DIM)

    xf = _ln(xf[0], fp["ln_g"], fp["ln_b"])
    lw = {k: flmats[k][...][None] for k in _LSTM_MATS}
    lw["b0"] = fp["fb0"][:, None, :]
    lw["b1"] = fp["fb1"][:, None, :]
    hfin = _bilstm2(xf[None], lw, DIM, f0_ref, f1_ref)[0]
    out = jnp.dot(hfin, head_w[...],
                  preferred_element_type=jnp.float32) + fp["head_b"]
    out_ref[...] = out[:, :NUM_CLASSES].reshape(B, T, NUM_CLASSES)


def kernel(p00, p01, p02, p03, p04, p05, p06, p07, p08, p09, p10, p11, p12,
           p13, p14, p15, p16, p17, p18, p19, p20, p21, p22, p23, p24, p25,
           p26, p27, p28, p29, p30, p31, p32, p33, p34, p35, p36, p37, p38,
           p39, p40, p41, p42, p43, p44, p45, p46, p47, p48, p49, p50, p51,
           p52, p53, p54, p55, p56, p57, p58, p59, p60, p61, p62, p63, p64,
           p65, p66, p67, p68, p69, p70, p71, p72, p73, p74, p75, p76, p77,
           p78, p79, p80, p81, p82, p83, p84, p85, p86, p87, p88, p89, p90,
           imu, keypoint, e4acc, bbox, ht, printer):
    del e4acc
    leaves = [p00, p01, p02, p03, p04, p05, p06, p07, p08, p09, p10, p11,
              p12, p13, p14, p15, p16, p17, p18, p19, p20, p21, p22, p23,
              p24, p25, p26, p27, p28, p29, p30, p31, p32, p33, p34, p35,
              p36, p37, p38, p39, p40, p41, p42, p43, p44, p45, p46, p47,
              p48, p49, p50, p51, p52, p53, p54, p55, p56, p57, p58, p59,
              p60, p61, p62, p63, p64, p65, p66, p67, p68, p69, p70, p71,
              p72, p73, p74, p75, p76, p77, p78, p79, p80, p81, p82, p83,
              p84, p85, p86, p87, p88, p89, p90]
    treedef = jax.tree_util.tree_structure(_tree_template())
    params = jax.tree_util.tree_unflatten(treedef, leaves)

    st = params["streams"]
    blk = st["blocks"][0]
    inner = st["inner"]
    fus = params["fusion"]
    fblk = params["layers"][0]
    fin = params["final"]

    spack_parts = {"emb_b": st["emb_b"], **blk,
                   **{k: inner[k] for k in ("b0", "b1", "p1_b", "p2_b")}}
    spack = jnp.concatenate(
        [spack_parts[k].reshape(NUM_STREAMS, 1, -1) for k, _ in _SPACK],
        axis=2)
    fpack_parts = {**fblk, "fus_b1": fus["b1"], "fus_b2": fus["b2"],
                   "ln_g": fin["ln_g"], "ln_b": fin["ln_b"],
                   "fb0": fin["lstm"]["b0"], "fb1": fin["lstm"]["b1"],
                   "head_b": fin["head_b"]}
    fpack = jnp.concatenate(
        [fpack_parts[k].reshape(1, -1) for k, _ in _FPACK], axis=1)

    ins = [imu, keypoint, bbox, ht, printer,
           st["emb_w"], st["pos"], params["emb_ht"], params["emb_printer"],
           spack, fpack]
    ins += [blk[k] for k in _CONF_MATS]
    ins += [inner[k] for k in _LSTM_MATS] + [inner["p1_w"], inner["p2_w"]]
    ins += [fus["w1"], fus["w2"]]
    ins += [fblk[k] for k in _CONF_MATS]
    ins += [fin["lstm"][k] for k in _LSTM_MATS]
    ins += [fin["head_w"]]

    vmem = pl.BlockSpec(memory_space=pltpu.MemorySpace.VMEM)
    out, inner_out = pl.pallas_call(
        _mega_kernel,
        in_specs=[vmem] * len(ins),
        out_specs=(vmem, vmem),
        out_shape=(jax.ShapeDtypeStruct((B, T, NUM_CLASSES), jnp.float32),
                   jax.ShapeDtypeStruct((B, T, NUM_CLASSES), jnp.float32)),
        scratch_shapes=[pltpu.VMEM((NUM_STREAMS, B, T, 2 * SD), jnp.float32),
                        pltpu.VMEM((NUM_STREAMS, B, T, 2 * SD), jnp.float32),
                        pltpu.VMEM((1, B, T, 2 * DIM), jnp.float32),
                        pltpu.VMEM((1, B, T, 2 * DIM), jnp.float32)],
    )(*ins)
    return out, inner_out


# speedup vs baseline: 1.2998x; 1.2998x over previous
import math

import jax
import jax.numpy as jnp
from jax.experimental import pallas as pl
from jax.experimental.pallas import tpu as pltpu

B, T = 2, 8
M = B * T
DEPTH = 1
HEADS = 4
NUM_CLASSES = 11
C_PAD = 128
SD = 32
HT_DIM, PR_DIM = 16, 16
DIM = 128
NUM_STREAMS = 3
IMU_IN, KP_IN, BBOX_IN = 24, 32, 8
CONV_K = 5
CONV_PAD = 2
LN_EPS = 1e-5

_CONF_KEYS = ("ff1_g", "ff1_b", "ff1_w1", "ff1_b1", "ff1_w2", "ff1_b2",
              "at_g", "at_b", "w_qkv", "b_qkv", "w_o", "b_o",
              "cv_g", "cv_b", "pw1_w", "pw1_b", "dw_w", "dw_s", "dw_sh",
              "pw2_w", "pw2_b",
              "ff2_g", "ff2_b", "ff2_w1", "ff2_b1", "ff2_w2", "ff2_b2",
              "lno_g", "lno_b")
_LSTM_KEYS = ("wih0", "b0", "whh0f", "whh0b", "wih1", "b1", "whh1f", "whh1b")
_INNER_KEYS = _LSTM_KEYS + ("p1_w", "p1_b", "p2_w", "p2_b")

_CF_MATS = ("ff1_w1", "ff1_w2", "w_qkv", "w_o", "pw1_w", "pw2_w",
            "ff2_w1", "ff2_w2")
_CONF_SMALLS = tuple(k for k in _CONF_KEYS if k not in _CF_MATS)
_IN_MATS = ("wih0", "whh0f", "whh0b", "wih1", "whh1f", "whh1b",
            "p1_w", "p2_w")
_FL_MATS = ("wih0", "whh0f", "whh0b", "wih1", "whh1f", "whh1b")
_N_MANUAL = len(_IN_MATS) + 2 + len(_CF_MATS) + len(_FL_MATS) + 1


def _tree_template():
    conf = lambda: {k: 0 for k in _CONF_KEYS}
    inner = {k: 0 for k in _INNER_KEYS}
    streams = {"emb_w": 0, "emb_b": 0, "pos": 0,
               "blocks": [conf() for _ in range(DEPTH)], "inner": inner}
    return {"streams": streams, "emb_ht": 0, "emb_printer": 0,
            "fusion": {"w1": 0, "b1": 0, "w2": 0, "b2": 0},
            "layers": [conf() for _ in range(DEPTH)],
            "final": {"ln_g": 0, "ln_b": 0,
                      "lstm": {k: 0 for k in _LSTM_KEYS},
                      "head_w": 0, "head_b": 0}}


def _silu(x):
    return x * jax.nn.sigmoid(x)


def _ln(x, g, b):
    mu = jnp.mean(x, axis=-1, keepdims=True)
    var = jnp.mean(jnp.square(x - mu), axis=-1, keepdims=True)
    return (x - mu) * jax.lax.rsqrt(var + LN_EPS) * g + b


def _bmm(x, w):
    return jnp.einsum("smd,sdk->smk", x, w,
                      preferred_element_type=jnp.float32)


def _bmmh(x, w):
    return jnp.einsum("smd,sdk->smk", x.astype(jnp.bfloat16),
                      w.astype(jnp.bfloat16),
                      preferred_element_type=jnp.float32)


def _cell(g, c, hd):
    sg = jax.nn.sigmoid(g)
    tg = jnp.tanh(g[..., 2 * hd:3 * hd])
    c = sg[..., 1 * hd:2 * hd] * c + sg[..., 0 * hd:1 * hd] * tg
    return sg[..., 3 * hd:4 * hd] * jnp.tanh(c), c


def _bidir(xg, whf, whb, out_ref, hd):
    s = xg.shape[0]
    whf = whf.astype(jnp.bfloat16)
    whb = whb.astype(jnp.bfloat16)
    hf = jnp.zeros((s, B, hd), jnp.float32)
    cf = hf
    hb = hf
    cb = hf
    for u in range(T):
        v = T - 1 - u
        gf = xg[:, :, u, :4 * hd] + jnp.einsum(
            "sbh,shk->sbk", hf.astype(jnp.bfloat16), whf,
            preferred_element_type=jnp.float32)
        gb = xg[:, :, v, 4 * hd:] + jnp.einsum(
            "sbh,shk->sbk", hb.astype(jnp.bfloat16), whb,
            preferred_element_type=jnp.float32)
        hf, cf = _cell(gf, cf, hd)
        hb, cb = _cell(gb, cb, hd)
        out_ref[:, :, u:u + 1, 0:hd] = hf[:, :, None, :]
        out_ref[:, :, v:v + 1, hd:2 * hd] = hb[:, :, None, :]


def _bilstm2(x, w, hd, s0_ref, s1_ref):
    s = x.shape[0]
    xg0 = (_bmmh(x, w["wih0"]) + w["b0"]).reshape(s, B, T, 8 * hd)
    _bidir(xg0, w["whh0f"], w["whh0b"], s0_ref, hd)
    h0 = s0_ref[...].reshape(s, M, 2 * hd)
    xg1 = (_bmmh(h0, w["wih1"]) + w["b1"]).reshape(s, B, T, 8 * hd)
    _bidir(xg1, w["whh1f"], w["whh1b"], s1_ref, hd)
    return s1_ref[...].reshape(s, M, 2 * hd)


def _conformer(x, w, d):
    s = x.shape[0]
    dh = d // HEADS
    scale = 1.0 / math.sqrt(dh)

    y = _ln(x, w["ff1_g"], w["ff1_b"])
    y = _silu(_bmm(y, w["ff1_w1"]) + w["ff1_b1"])
    x = x + 0.5 * (_bmm(y, w["ff1_w2"]) + w["ff1_b2"])

    y = _ln(x, w["at_g"], w["at_b"])
    qkv = (_bmm(y, w["w_qkv"]) + w["b_qkv"]).reshape(s * B, T, 3 * d)
    q = jnp.concatenate([qkv[..., h * dh:(h + 1) * dh]
                         for h in range(HEADS)], axis=0)
    k = jnp.concatenate([qkv[..., d + h * dh:d + (h + 1) * dh]
                         for h in range(HEADS)], axis=0)
    v = jnp.concatenate([qkv[..., 2 * d + h * dh:2 * d + (h + 1) * dh]
                         for h in range(HEADS)], axis=0)
    sc = jnp.einsum("btd,bud->btu", q, k,
                    preferred_element_type=jnp.float32) * scale
    sc = sc - jnp.max(sc, axis=-1, keepdims=True)
    p = jnp.exp(sc)
    p = p / jnp.sum(p, axis=-1, keepdims=True)
    o = jnp.einsum("btu,bud->btd", p, v,
                   preferred_element_type=jnp.float32)
    att = jnp.concatenate([o[h * s * B:(h + 1) * s * B]
                           for h in range(HEADS)], axis=-1).reshape(s, M, d)
    x = x + _bmm(att, w["w_o"]) + w["b_o"]

    y = _ln(x, w["cv_g"], w["cv_b"])
    y = _bmm(y, w["pw1_w"]) + w["pw1_b"]
    y = y[..., :d] * jax.nn.sigmoid(y[..., d:])
    y4 = y.reshape(s, B, T, d)
    zpad = jnp.zeros((s, B, CONV_PAD, d), jnp.float32)
    yp = jnp.concatenate([zpad, y4, zpad], axis=2)
    acc = jnp.zeros((s, B, T, d), jnp.float32)
    for kk in range(CONV_K):
        acc = acc + yp[:, :, kk:kk + T, :] * w["dw_w"][:, None, kk:kk + 1, :]
    y4 = _silu(acc * w["dw_s"][:, None] + w["dw_sh"][:, None])
    x = x + _bmm(y4.reshape(s, M, d), w["pw2_w"]) + w["pw2_b"]

    y = _ln(x, w["ff2_g"], w["ff2_b"])
    y = _silu(_bmm(y, w["ff2_w1"]) + w["ff2_b1"])
    x = x + 0.5 * (_bmm(y, w["ff2_w2"]) + w["ff2_b2"])

    return _ln(x, w["lno_g"], w["lno_b"])


def _mega_kernel(*refs):
    (imu_r, kp_r, bb_r, htf_r, prf_r,
     emb_w, emb_b, pos, eht, epr) = refs[:10]
    i = 10
    sblk = {k: refs[i + j] for j, k in enumerate(_CONF_KEYS)}
    i += len(_CONF_KEYS)
    in_b0, in_b1, in_p1b, in_p2b = refs[i:i + 4]
    fus_b1, fus_b2 = refs[i + 4], refs[i + 6 - 1]
    i += 6
    fsmall = {k: refs[i + j] for j, k in enumerate(_CONF_SMALLS)}
    i += len(_CONF_SMALLS)
    fin_g, fin_b, fl_b0, fl_b1, head_b = refs[i:i + 5]
    i += 5
    hbm = refs[i:i + _N_MANUAL]
    i += _N_MANUAL
    out_ref, inner_ref = refs[i], refs[i + 1]
    s0_ref, s1_ref, f0_ref, f1_ref = refs[i + 2:i + 6]
    bufs = refs[i + 6:i + 6 + _N_MANUAL]
    sems = refs[i + 6 + _N_MANUAL]

    cps = [pltpu.make_async_copy(hbm[j], bufs[j], sems.at[j])
           for j in range(_N_MANUAL)]
    for c in cps:
        c.start()

    pos_t = pos[:, :T, :]
    pos_m = jnp.concatenate([pos_t, pos_t], axis=1)

    def _embed(chunks, w_rows, widths):
        acc = jnp.zeros((M, SD), jnp.float32)
        off = 0
        for ch, wd in zip(chunks, widths):
            acc = acc + jnp.dot(ch.reshape(M, wd), w_rows[off:off + wd, :],
                                preferred_element_type=jnp.float32)
            off += wd
        return acc

    x0 = _embed([imu_r[:, :, f, :] for f in range(4)],
                emb_w[0], [6] * 4)
    x1 = _embed([kp_r[:, :, i, j, :] for i in range(2) for j in range(2)],
                emb_w[1], [8] * 4)
    x2 = _embed([bb_r[:, :, i, :] for i in range(2)],
                emb_w[2], [4] * 2)
    x = jnp.stack([x0, x1, x2], axis=0) + emb_b[...] + pos_m

    sw = {k: sblk[k][...] for k in _CONF_KEYS}
    x = _conformer(x, sw, SD)
    for c in cps[:8]:
        c.wait()
    iw = {k: bufs[j][...] for j, k in enumerate(_IN_MATS)}
    iw.update(b0=in_b0[...], b1=in_b1[...], p1_b=in_p1b[...],
              p2_b=in_p2b[...])
    h = _bilstm2(x, iw, SD, s0_ref, s1_ref)
    logits = _bmm(h, iw["p1_w"]) + iw["p1_b"]
    lane = jax.lax.broadcasted_iota(jnp.int32, logits.shape, 2)
    valid = lane < NUM_CLASSES
    mx = jnp.max(jnp.where(valid, logits, -jnp.inf), axis=-1, keepdims=True)
    e = jnp.where(valid, jnp.exp(logits - mx), 0.0)
    p = e / jnp.sum(e, axis=-1, keepdims=True)
    x = x + _bmm(p, iw["p2_w"]) + iw["p2_b"]
    inner_ref[...] = jnp.mean(logits, axis=0)[:, :NUM_CLASSES].reshape(
        B, T, NUM_CLASSES)

    bsel = (jax.lax.broadcasted_iota(jnp.int32, (M, B), 0) // T
            == jax.lax.broadcasted_iota(jnp.int32, (M, B), 1)
            ).astype(jnp.float32)
    tmask = (jax.lax.broadcasted_iota(jnp.int32, (M, T), 0) % T
             == jax.lax.broadcasted_iota(jnp.int32, (M, T), 1)
             ).astype(jnp.float32)
    htf = jnp.sum(jnp.dot(bsel, htf_r[...].astype(jnp.float32),
                          preferred_element_type=jnp.float32) * tmask,
                  axis=1, keepdims=True)
    prf = jnp.sum(jnp.dot(bsel, prf_r[...].astype(jnp.float32),
                          preferred_element_type=jnp.float32) * tmask,
                  axis=1, keepdims=True)
    e0, e1 = eht[0:1, :], eht[1:2, :]
    x_ht = e0 + htf * (e1 - e0)
    q0, q1 = epr[0:1, :], epr[1:2, :]
    x_pr = q0 + prf * (q1 - q0)

    xf = jnp.concatenate([x[0], x[1], x_ht, x_pr, x[2]], axis=-1)
    cps[8].wait()
    cps[9].wait()
    y = _silu(jnp.dot(xf, bufs[8][...],
                      preferred_element_type=jnp.float32) + fus_b1[...])
    xf = jnp.dot(y, bufs[9][...],
                 preferred_element_type=jnp.float32) + fus_b2[...]

    for c in cps[10:18]:
        c.wait()
    fw = {k: bufs[10 + j][...] for j, k in enumerate(_CF_MATS)}
    fw.update({k: fsmall[k][...] for k in _CONF_SMALLS})
    xf = _conformer(xf[None], fw, DIM)

    for c in cps[18:]:
        c.wait()
    xf = _ln(xf[0], fin_g[...], fin_b[...])
    lw = {k: bufs[18 + j][...][None] for j, k in enumerate(_FL_MATS)}
    lw["b0"] = fl_b0[...][None]
    lw["b1"] = fl_b1[...][None]
    hfin = _bilstm2(xf[None], lw, DIM, f0_ref, f1_ref)[0]
    out = jnp.dot(hfin, bufs[24][...],
                  preferred_element_type=jnp.float32) + head_b[...]
    out_ref[...] = out[:, :NUM_CLASSES].reshape(B, T, NUM_CLASSES)


def kernel(p00, p01, p02, p03, p04, p05, p06, p07, p08, p09, p10, p11, p12,
           p13, p14, p15, p16, p17, p18, p19, p20, p21, p22, p23, p24, p25,
           p26, p27, p28, p29, p30, p31, p32, p33, p34, p35, p36, p37, p38,
           p39, p40, p41, p42, p43, p44, p45, p46, p47, p48, p49, p50, p51,
           p52, p53, p54, p55, p56, p57, p58, p59, p60, p61, p62, p63, p64,
           p65, p66, p67, p68, p69, p70, p71, p72, p73, p74, p75, p76, p77,
           p78, p79, p80, p81, p82, p83, p84, p85, p86, p87, p88, p89, p90,
           imu, keypoint, e4acc, bbox, ht, printer):
    del e4acc
    leaves = [p00, p01, p02, p03, p04, p05, p06, p07, p08, p09, p10, p11,
              p12, p13, p14, p15, p16, p17, p18, p19, p20, p21, p22, p23,
              p24, p25, p26, p27, p28, p29, p30, p31, p32, p33, p34, p35,
              p36, p37, p38, p39, p40, p41, p42, p43, p44, p45, p46, p47,
              p48, p49, p50, p51, p52, p53, p54, p55, p56, p57, p58, p59,
              p60, p61, p62, p63, p64, p65, p66, p67, p68, p69, p70, p71,
              p72, p73, p74, p75, p76, p77, p78, p79, p80, p81, p82, p83,
              p84, p85, p86, p87, p88, p89, p90]
    treedef = jax.tree_util.tree_structure(_tree_template())
    params = jax.tree_util.tree_unflatten(treedef, leaves)

    st = params["streams"]
    blk = st["blocks"][0]
    inner = st["inner"]
    fus = params["fusion"]
    fblk = params["layers"][0]
    fin = params["final"]

    auto_ins = [imu, keypoint, bbox, ht, printer,
                st["emb_w"], st["emb_b"], st["pos"],
                params["emb_ht"], params["emb_printer"]]
    auto_ins += [blk[k] for k in _CONF_KEYS]
    auto_ins += [inner["b0"], inner["b1"], inner["p1_b"], inner["p2_b"]]
    auto_ins += [fus["b1"], fus["b2"]]
    auto_ins += [fblk[k] for k in _CONF_SMALLS]
    auto_ins += [fin["ln_g"], fin["ln_b"],
                 fin["lstm"]["b0"], fin["lstm"]["b1"], fin["head_b"]]
    manual = ([inner[k] for k in _IN_MATS] + [fus["w1"], fus["w2"]]
              + [fblk[k] for k in _CF_MATS]
              + [fin["lstm"][k] for k in _FL_MATS] + [fin["head_w"]])

    vmem = pl.BlockSpec(memory_space=pltpu.MemorySpace.VMEM)
    hbm = pl.BlockSpec(memory_space=pltpu.MemorySpace.HBM)
    out, inner_out = pl.pallas_call(
        _mega_kernel,
        in_specs=[vmem] * len(auto_ins) + [hbm] * len(manual),
        out_specs=(vmem, vmem),
        out_shape=(jax.ShapeDtypeStruct((B, T, NUM_CLASSES), jnp.float32),
                   jax.ShapeDtypeStruct((B, T, NUM_CLASSES), jnp.float32)),
        scratch_shapes=([pltpu.VMEM((NUM_STREAMS, B, T, 2 * SD), jnp.float32),
                         pltpu.VMEM((NUM_STREAMS, B, T, 2 * SD), jnp.float32),
                         pltpu.VMEM((1, B, T, 2 * DIM), jnp.float32),
                         pltpu.VMEM((1, B, T, 2 * DIM), jnp.float32)]
                        + [pltpu.VMEM(a.shape, jnp.float32) for a in manual]
                        + [pltpu.SemaphoreType.DMA((_N_MANUAL,))]),
    )(*(auto_ins + manual))
    return out, inner_out
